# baseline (speedup 1.0000x reference)
import functools

import jax
import jax.numpy as jnp
from jax import lax
from jax.experimental import pallas as pl
from jax.experimental.pallas import tpu as pltpu
from jax.experimental.pallas import tpu_sc as plsc

NE = 160000
NN = 10000
NNP = 10240
EPW = NE // 32
ECH = 440
ECT = EPW - 11 * ECH
GCH = 312
GCT = EPW - 16 * GCH
HP = 128
NPT = NNP // 16
_SC_MESH = plsc.VectorSubcoreMesh(core_axis_name="c", subcore_axis_name="s")


def _gather_body(h_hbm, src_hbm, out_hbm, idx_v, rows_v, idx_t, rows_t, sem):
    wid = lax.axis_index("c") * 16 + lax.axis_index("s")
    for ch in range(16):
        base = wid * EPW + ch * GCH
        pltpu.sync_copy(src_hbm.at[pl.ds(base, GCH)], idx_v)
        pltpu.async_copy(h_hbm.at[idx_v], rows_v, sem).wait()
        pltpu.sync_copy(rows_v, out_hbm.at[pl.ds(base, GCH)])
    base = wid * EPW + 16 * GCH
    pltpu.sync_copy(src_hbm.at[pl.ds(base, GCT)], idx_t)
    pltpu.async_copy(h_hbm.at[idx_t], rows_t, sem).wait()
    pltpu.sync_copy(rows_t, out_hbm.at[pl.ds(base, GCT)])


def _scatter_body(msg_hbm, dst_hbm, zeros_hbm, out_hbm, idx_v, vals_v, idx_t, vals_t, shared, sem):
    c = lax.axis_index("c")
    s = lax.axis_index("s")
    pltpu.sync_copy(zeros_hbm.at[pl.ds(s * NPT, NPT)], shared.at[pl.ds(s * NPT, NPT)])
    plsc.subcore_barrier()
    wid = c * 16 + s
    for ch in range(11):
        base = wid * EPW + ch * ECH
        pltpu.sync_copy(dst_hbm.at[pl.ds(base, ECH)], idx_v)
        pltpu.sync_copy(msg_hbm.at[pl.ds(base, ECH)], vals_v)
        pltpu.sync_copy(vals_v, shared.at[idx_v], add=True)
    base = wid * EPW + 11 * ECH
    pltpu.sync_copy(dst_hbm.at[pl.ds(base, ECT)], idx_t)
    pltpu.sync_copy(msg_hbm.at[pl.ds(base, ECT)], vals_t)
    pltpu.sync_copy(vals_t, shared.at[idx_t], add=True)
    plsc.subcore_barrier()
    pltpu.sync_copy(shared.at[pl.ds(s * NPT, NPT)],
                    out_hbm.at[c, pl.ds(s * NPT, NPT)])


H = 32

_sc_gather = pl.kernel(
    _gather_body,
    out_type=jax.ShapeDtypeStruct((NE, HP), jnp.float32),
    mesh=_SC_MESH,
    scratch_types=[
        pltpu.VMEM((GCH,), jnp.int32),
        pltpu.VMEM((GCH, HP), jnp.float32),
        pltpu.VMEM((GCT,), jnp.int32),
        pltpu.VMEM((GCT, HP), jnp.float32),
        pltpu.SemaphoreType.DMA,
    ],
)

_sc_scatter = pl.kernel(
    _scatter_body,
    out_type=jax.ShapeDtypeStruct((2, NNP, H), jnp.float32),
    mesh=_SC_MESH,
    scratch_types=[
        pltpu.VMEM((ECH,), jnp.int32),
        pltpu.VMEM((ECH, H), jnp.float32),
        pltpu.VMEM((ECT,), jnp.int32),
        pltpu.VMEM((ECT, H), jnp.float32),
        pltpu.VMEM_SHARED((NNP, H), jnp.float32),
        pltpu.SemaphoreType.DMA,
    ],
)
BN = 3200


def _msg_body(hs_ref, eeT_ref, w2_ref, out_ref):
    hsT = hs_ref[:, :H].astype(jnp.bfloat16).T
    eeT = eeT_ref[...]
    a = jnp.repeat(hsT, H, axis=0) * jnp.tile(eeT, (H, 1))
    a = jnp.concatenate([a, hsT], axis=0)
    out_ref[...] = jax.lax.dot_general(
        a, w2_ref[...], (((0,), (0,)), ((), ())),
        preferred_element_type=jnp.float32)


def _msg_matmul(hs, eeT, w2b):
    e = hs.shape[0]
    grid = (e // BN,)
    return pl.pallas_call(
        _msg_body,
        grid=grid,
        in_specs=[
            pl.BlockSpec((BN, HP), lambda j: (j, 0)),
            pl.BlockSpec((H, BN), lambda j: (0, j)),
            pl.BlockSpec((H * H + H, H), lambda j: (0, 0)),
        ],
        out_specs=pl.BlockSpec((BN, H), lambda j: (j, 0)),
        out_shape=jax.ShapeDtypeStruct((e, H), jnp.float32),
    )(hs, eeT, w2b)


GBM = 2048


def _gru_body(agg_ref, invd_ref, h_ref, wih_ref, whh_ref, bih_ref, bhh_ref, out_ref):
    m = jax.nn.relu((agg_ref[0] + agg_ref[1]) * invd_ref[...])
    h = h_ref[:, :H]
    gi = jax.lax.dot_general(m, wih_ref[...], (((1,), (1,)), ((), ())),
                             preferred_element_type=jnp.float32) + bih_ref[...]
    gh = jax.lax.dot_general(h, whh_ref[...], (((1,), (1,)), ((), ())),
                             preferred_element_type=jnp.float32) + bhh_ref[...]
    r = jax.nn.sigmoid(gi[:, :H] + gh[:, :H])
    z = jax.nn.sigmoid(gi[:, H:2 * H] + gh[:, H:2 * H])
    n = jnp.tanh(gi[:, 2 * H:] + r * gh[:, 2 * H:])
    hn = (1.0 - z) * n + z * h
    out_ref[...] = jnp.concatenate(
        [hn, jnp.zeros((GBM, HP - H), jnp.float32)], axis=1)


def _gru_layer(agg, invd, h, wih, whh, bih, bhh):
    grid = (NNP // GBM,)
    return pl.pallas_call(
        _gru_body,
        grid=grid,
        in_specs=[
            pl.BlockSpec((2, GBM, H), lambda j: (0, j, 0)),
            pl.BlockSpec((GBM, 1), lambda j: (j, 0)),
            pl.BlockSpec((GBM, HP), lambda j: (j, 0)),
            pl.BlockSpec((3 * H, H), lambda j: (0, 0)),
            pl.BlockSpec((3 * H, H), lambda j: (0, 0)),
            pl.BlockSpec((1, 3 * H), lambda j: (0, 0)),
            pl.BlockSpec((1, 3 * H), lambda j: (0, 0)),
        ],
        out_specs=pl.BlockSpec((GBM, HP), lambda j: (j, 0)),
        out_shape=jax.ShapeDtypeStruct((NNP, HP), jnp.float32),
    )(agg, invd, h, wih, whh, bih, bhh)


def _gru_cell(x, h, Wih, Whh, bih, bhh):
    gi = x @ Wih.T + bih
    gh = h @ Whh.T + bhh
    i_r, i_z, i_n = jnp.split(gi, 3, axis=-1)
    h_r, h_z, h_n = jnp.split(gh, 3, axis=-1)
    r = jax.nn.sigmoid(i_r + h_r)
    z = jax.nn.sigmoid(i_z + h_z)
    n = jnp.tanh(i_n + r * h_n)
    return (1.0 - z) * n + z * h


def kernel(x_node, x_edge, edge_index, node_W, node_b, edge_W, edge_b,
           en_W1, en_b1, en_W2, en_b2, gru_Wih, gru_Whh, gru_bih, gru_bhh):
    src = edge_index[0]
    dst = edge_index[1]
    n_nodes = x_node.shape[0]

    h = x_node @ node_W.T + node_b
    he = x_edge @ edge_W.T + edge_b
    ee = jax.nn.relu(he @ en_W1.T + en_b1)
    eeT = ee.T.astype(jnp.bfloat16)

    w2flat = en_W2.reshape(H, H, H).transpose(0, 2, 1).reshape(H * H, H)
    b2r = en_b2.reshape(H, H)
    w2b = jnp.concatenate([w2flat, b2r], axis=0).astype(jnp.bfloat16)

    bih = gru_bih[None, :]
    bhh = gru_bhh[None, :]
    zeros = jnp.zeros((NNP, H), jnp.float32)

    deg = jax.ops.segment_sum(jnp.ones_like(dst, jnp.float32), dst,
                              num_segments=n_nodes)
    inv_deg = (1.0 / jnp.maximum(deg, 1.0))[:, None]
    inv_deg = jnp.concatenate(
        [inv_deg, jnp.zeros((NNP - NN, 1), jnp.float32)], axis=0)

    hp0 = jnp.zeros((NNP, HP), jnp.float32).at[:NN, :H].set(h)

    def layer(hcur, _):
        hsb = _sc_gather(hcur, src)
        msg = _msg_matmul(hsb, eeT, w2b)
        agg0 = jax.ops.segment_sum(msg.astype(jnp.bfloat16), dst,
                                   num_segments=NNP).astype(jnp.float32)
        agg = jnp.stack([agg0, jnp.zeros_like(agg0)])
        hnew = _gru_layer(agg, inv_deg, hcur, gru_Wih, gru_Whh, bih, bhh)
        return hnew, None

    hp, _ = lax.scan(layer, hp0, None, length=3)
    return hp[:NN, :H]


# speedup vs baseline: 1.9036x; 1.9036x over previous
import jax
import jax.numpy as jnp
from jax import lax
from jax.experimental import pallas as pl
from jax.experimental.pallas import tpu as pltpu
from jax.experimental.pallas import tpu_sc as plsc

NE = 160000
NN = 10000
NNP = 10240
EPW = NE // 32
GCH = 312
GCT = EPW - 16 * GCH
HP = 128
NPT = NNP // 16
_SC_MESH = plsc.VectorSubcoreMesh(core_axis_name="c", subcore_axis_name="s")


def _gather_body(h_hbm, src_hbm, out_hbm, idx_v, rows_v, idx_t, rows_t, sem):
    wid = lax.axis_index("c") * 16 + lax.axis_index("s")
    for ch in range(16):
        base = wid * EPW + ch * GCH
        pltpu.sync_copy(src_hbm.at[pl.ds(base, GCH)], idx_v)
        pltpu.async_copy(h_hbm.at[idx_v], rows_v, sem).wait()
        pltpu.sync_copy(rows_v, out_hbm.at[pl.ds(base, GCH)])
    base = wid * EPW + 16 * GCH
    pltpu.sync_copy(src_hbm.at[pl.ds(base, GCT)], idx_t)
    pltpu.async_copy(h_hbm.at[idx_t], rows_t, sem).wait()
    pltpu.sync_copy(rows_t, out_hbm.at[pl.ds(base, GCT)])


H = 32

_sc_gather = pl.kernel(
    _gather_body,
    out_type=jax.ShapeDtypeStruct((NE, HP), jnp.float32),
    mesh=_SC_MESH,
    scratch_types=[
        pltpu.VMEM((GCH,), jnp.int32),
        pltpu.VMEM((GCH, HP), jnp.float32),
        pltpu.VMEM((GCT,), jnp.int32),
        pltpu.VMEM((GCT, HP), jnp.float32),
        pltpu.SemaphoreType.DMA,
    ],
)

BN = 3200


def _msg_body(hs_ref, eeT_ref, w2_ref, out_ref):
    hsT = hs_ref[:, :H].astype(jnp.bfloat16).T
    eeT = eeT_ref[...]
    a = jnp.repeat(hsT, H, axis=0) * jnp.tile(eeT, (H, 1))
    a = jnp.concatenate([a, hsT], axis=0)
    out_ref[...] = jax.lax.dot_general(
        a, w2_ref[...], (((0,), (0,)), ((), ())),
        preferred_element_type=jnp.float32)


def _msg_matmul(hs, eeT, w2b):
    e = hs.shape[0]
    grid = (e // BN,)
    return pl.pallas_call(
        _msg_body,
        grid=grid,
        in_specs=[
            pl.BlockSpec((BN, HP), lambda j: (j, 0)),
            pl.BlockSpec((H, BN), lambda j: (0, j)),
            pl.BlockSpec((H * H + H, H), lambda j: (0, 0)),
        ],
        out_specs=pl.BlockSpec((BN, H), lambda j: (j, 0)),
        out_shape=jax.ShapeDtypeStruct((e, H), jnp.float32),
    )(hs, eeT, w2b)


GBM = 2048


def _gru_body(agg_ref, invd_ref, h_ref, wih_ref, whh_ref, bih_ref, bhh_ref, out_ref):
    m = jax.nn.relu(agg_ref[...] * invd_ref[...])
    h = h_ref[:, :H]
    gi = jax.lax.dot_general(m, wih_ref[...], (((1,), (1,)), ((), ())),
                             preferred_element_type=jnp.float32) + bih_ref[...]
    gh = jax.lax.dot_general(h, whh_ref[...], (((1,), (1,)), ((), ())),
                             preferred_element_type=jnp.float32) + bhh_ref[...]
    r = jax.nn.sigmoid(gi[:, :H] + gh[:, :H])
    z = jax.nn.sigmoid(gi[:, H:2 * H] + gh[:, H:2 * H])
    n = jnp.tanh(gi[:, 2 * H:] + r * gh[:, 2 * H:])
    hn = (1.0 - z) * n + z * h
    out_ref[...] = jnp.concatenate(
        [hn, jnp.zeros((GBM, HP - H), jnp.float32)], axis=1)


def _gru_layer(agg, invd, h, wih, whh, bih, bhh):
    grid = (NNP // GBM,)
    return pl.pallas_call(
        _gru_body,
        grid=grid,
        in_specs=[
            pl.BlockSpec((GBM, H), lambda j: (j, 0)),
            pl.BlockSpec((GBM, 1), lambda j: (j, 0)),
            pl.BlockSpec((GBM, HP), lambda j: (j, 0)),
            pl.BlockSpec((3 * H, H), lambda j: (0, 0)),
            pl.BlockSpec((3 * H, H), lambda j: (0, 0)),
            pl.BlockSpec((1, 3 * H), lambda j: (0, 0)),
            pl.BlockSpec((1, 3 * H), lambda j: (0, 0)),
        ],
        out_specs=pl.BlockSpec((GBM, HP), lambda j: (j, 0)),
        out_shape=jax.ShapeDtypeStruct((NNP, HP), jnp.float32),
    )(agg, invd, h, wih, whh, bih, bhh)


def kernel(x_node, x_edge, edge_index, node_W, node_b, edge_W, edge_b,
           en_W1, en_b1, en_W2, en_b2, gru_Wih, gru_Whh, gru_bih, gru_bhh):
    src = edge_index[0]
    dst = edge_index[1]
    n_nodes = x_node.shape[0]

    h = x_node @ node_W.T + node_b
    he = x_edge @ edge_W.T + edge_b
    ee = jax.nn.relu(he @ en_W1.T + en_b1)
    eeT = ee.T.astype(jnp.bfloat16)

    w2flat = en_W2.reshape(H, H, H).transpose(0, 2, 1).reshape(H * H, H)
    b2r = en_b2.reshape(H, H)
    w2b = jnp.concatenate([w2flat, b2r], axis=0).astype(jnp.bfloat16)

    bih = gru_bih[None, :]
    bhh = gru_bhh[None, :]

    deg = jax.ops.segment_sum(jnp.ones_like(dst, jnp.float32), dst,
                              num_segments=n_nodes)
    inv_deg = (1.0 / jnp.maximum(deg, 1.0))[:, None]
    inv_deg = jnp.concatenate(
        [inv_deg, jnp.zeros((NNP - NN, 1), jnp.float32)], axis=0)

    hp0 = jnp.zeros((NNP, HP), jnp.float32).at[:NN, :H].set(h)

    def layer(hcur, _):
        hsb = _sc_gather(hcur, src)
        msg = _msg_matmul(hsb, eeT, w2b)
        agg = jax.ops.segment_sum(msg, dst, num_segments=NNP)
        hnew = _gru_layer(agg, inv_deg, hcur, gru_Wih, gru_Whh, bih, bhh)
        return hnew, None

    hp, _ = lax.scan(layer, hp0, None, length=3)
    return hp[:NN, :H]


# speedup vs baseline: 1.9365x; 1.0173x over previous
import jax
import jax.numpy as jnp
from jax import lax
from jax.experimental import pallas as pl
from jax.experimental.pallas import tpu as pltpu
from jax.experimental.pallas import tpu_sc as plsc

NE = 160000
NN = 10000
NNP = 10240
EPW = NE // 32
GCH = 624
GCT = EPW - 8 * GCH
HP = 128
NPT = NNP // 16
_SC_MESH = plsc.VectorSubcoreMesh(core_axis_name="c", subcore_axis_name="s")


def _gather_body(h_hbm, src_hbm, out_hbm, idx_v, rows_v, idx_t, rows_t, sem):
    wid = lax.axis_index("c") * 16 + lax.axis_index("s")
    for ch in range(8):
        base = wid * EPW + ch * GCH
        pltpu.sync_copy(src_hbm.at[pl.ds(base, GCH)], idx_v)
        pltpu.async_copy(h_hbm.at[idx_v], rows_v, sem).wait()
        pltpu.sync_copy(rows_v, out_hbm.at[pl.ds(base, GCH)])
    base = wid * EPW + 8 * GCH
    pltpu.sync_copy(src_hbm.at[pl.ds(base, GCT)], idx_t)
    pltpu.async_copy(h_hbm.at[idx_t], rows_t, sem).wait()
    pltpu.sync_copy(rows_t, out_hbm.at[pl.ds(base, GCT)])


H = 32

_sc_gather = pl.kernel(
    _gather_body,
    out_type=jax.ShapeDtypeStruct((NE, HP), jnp.float32),
    mesh=_SC_MESH,
    scratch_types=[
        pltpu.VMEM((GCH,), jnp.int32),
        pltpu.VMEM((GCH, HP), jnp.float32),
        pltpu.VMEM((GCT,), jnp.int32),
        pltpu.VMEM((GCT, HP), jnp.float32),
        pltpu.SemaphoreType.DMA,
    ],
)

BN = 6400


def _msg_body(hs_ref, eeT_ref, w2_ref, out_ref):
    hsT = hs_ref[:, :H].astype(jnp.bfloat16).T
    eeT = eeT_ref[...]
    a = jnp.repeat(hsT, H, axis=0) * jnp.tile(eeT, (H, 1))
    a = jnp.concatenate([a, hsT], axis=0)
    out_ref[...] = jax.lax.dot_general(
        a, w2_ref[...], (((0,), (0,)), ((), ())),
        preferred_element_type=jnp.float32)


def _msg_matmul(hs, eeT, w2b):
    e = hs.shape[0]
    grid = (e // BN,)
    return pl.pallas_call(
        _msg_body,
        grid=grid,
        in_specs=[
            pl.BlockSpec((BN, HP), lambda j: (j, 0)),
            pl.BlockSpec((H, BN), lambda j: (0, j)),
            pl.BlockSpec((H * H + H, H), lambda j: (0, 0)),
        ],
        out_specs=pl.BlockSpec((BN, H), lambda j: (j, 0)),
        out_shape=jax.ShapeDtypeStruct((e, H), jnp.float32),
    )(hs, eeT, w2b)


GBM = 2048


def _gru_body(agg_ref, invd_ref, h_ref, wih_ref, whh_ref, bih_ref, bhh_ref, out_ref):
    m = jax.nn.relu(agg_ref[...] * invd_ref[...])
    h = h_ref[:, :H]
    gi = jax.lax.dot_general(m, wih_ref[...], (((1,), (1,)), ((), ())),
                             preferred_element_type=jnp.float32) + bih_ref[...]
    gh = jax.lax.dot_general(h, whh_ref[...], (((1,), (1,)), ((), ())),
                             preferred_element_type=jnp.float32) + bhh_ref[...]
    r = jax.nn.sigmoid(gi[:, :H] + gh[:, :H])
    z = jax.nn.sigmoid(gi[:, H:2 * H] + gh[:, H:2 * H])
    n = jnp.tanh(gi[:, 2 * H:] + r * gh[:, 2 * H:])
    hn = (1.0 - z) * n + z * h
    out_ref[...] = jnp.concatenate(
        [hn, jnp.zeros((GBM, HP - H), jnp.float32)], axis=1)


def _gru_layer(agg, invd, h, wih, whh, bih, bhh):
    grid = (NNP // GBM,)
    return pl.pallas_call(
        _gru_body,
        grid=grid,
        in_specs=[
            pl.BlockSpec((GBM, H), lambda j: (j, 0)),
            pl.BlockSpec((GBM, 1), lambda j: (j, 0)),
            pl.BlockSpec((GBM, HP), lambda j: (j, 0)),
            pl.BlockSpec((3 * H, H), lambda j: (0, 0)),
            pl.BlockSpec((3 * H, H), lambda j: (0, 0)),
            pl.BlockSpec((1, 3 * H), lambda j: (0, 0)),
            pl.BlockSpec((1, 3 * H), lambda j: (0, 0)),
        ],
        out_specs=pl.BlockSpec((GBM, HP), lambda j: (j, 0)),
        out_shape=jax.ShapeDtypeStruct((NNP, HP), jnp.float32),
    )(agg, invd, h, wih, whh, bih, bhh)


def kernel(x_node, x_edge, edge_index, node_W, node_b, edge_W, edge_b,
           en_W1, en_b1, en_W2, en_b2, gru_Wih, gru_Whh, gru_bih, gru_bhh):
    src = edge_index[0]
    dst = edge_index[1]
    n_nodes = x_node.shape[0]

    h = x_node @ node_W.T + node_b
    he = x_edge @ edge_W.T + edge_b
    ee = jax.nn.relu(he @ en_W1.T + en_b1)
    eeT = ee.T.astype(jnp.bfloat16)

    w2flat = en_W2.reshape(H, H, H).transpose(0, 2, 1).reshape(H * H, H)
    b2r = en_b2.reshape(H, H)
    w2b = jnp.concatenate([w2flat, b2r], axis=0).astype(jnp.bfloat16)

    bih = gru_bih[None, :]
    bhh = gru_bhh[None, :]

    deg = jax.ops.segment_sum(jnp.ones_like(dst, jnp.float32), dst,
                              num_segments=n_nodes)
    inv_deg = (1.0 / jnp.maximum(deg, 1.0))[:, None]
    inv_deg = jnp.concatenate(
        [inv_deg, jnp.zeros((NNP - NN, 1), jnp.float32)], axis=0)

    hp0 = jnp.zeros((NNP, HP), jnp.float32).at[:NN, :H].set(h)

    def layer(hcur, _):
        hsb = _sc_gather(hcur, src)
        msg = _msg_matmul(hsb, eeT, w2b)
        agg = jax.ops.segment_sum(msg, dst, num_segments=NNP)
        hnew = _gru_layer(agg, inv_deg, hcur, gru_Wih, gru_Whh, bih, bhh)
        return hnew, None

    hp, _ = lax.scan(layer, hp0, None, length=3)
    return hp[:NN, :H]
